# Initial kernel scaffold; baseline (speedup 1.0000x reference)
import jax
import jax.numpy as jnp
from jax.experimental import pallas as pl


def kernel(X, Wv, bv, We, be, edge_index):
    raise NotImplementedError("write your pallas kernel here")


# speedup vs baseline: 11.1417x; 11.1417x over previous
import functools

import jax
import jax.numpy as jnp
from jax import lax
from jax.experimental import pallas as pl
from jax.experimental.pallas import tpu as pltpu
from jax.experimental.pallas import tpu_sc as plsc

NV = 10000
NE = 10000
NNZ = 320000
C = 128
HC = C // 2

NC = 2
NS = 16
NW = NC * NS
P = NNZ // NS
K1 = 80
NCH1 = P // K1
NBUF1 = 5
K2 = 125
NCH2 = P // K2
NBUF2 = 5
RPT = NE // NS
CW = 16

@functools.cache
def _mesh():
    return plsc.VectorSubcoreMesh(
        core_axis_name="c", subcore_axis_name="s",
        num_cores=NC, num_subcores=NS)


def _zero_rows(ref):
    rows, cols = ref.shape
    zrow = jnp.zeros((16,), jnp.float32)
    def body(r, carry):
        for c in range(cols // 16):
            ref[r, pl.ds(c * 16, 16)] = zrow
        return carry
    lax.fori_loop(0, rows, body, 0)


def _stripe_chunks(k):
    full = [(i * k, k) for i in range(RPT // k)]
    if RPT % k:
        full.append(((RPT // k) * k, RPT % k))
    return full


def _zero_stripe(acc, base, rows_v):
    for off, ln in _stripe_chunks(rows_v.shape[0]):
        pltpu.sync_copy(rows_v.at[pl.ds(0, ln)], acc.at[pl.ds(base + off, ln)])


def _copy_out_stripe(acc, base, rows_v, out_ref):
    for off, ln in _stripe_chunks(rows_v.shape[0]):
        pltpu.sync_copy(acc.at[pl.ds(base + off, ln)], rows_v.at[pl.ds(0, ln)])
        pltpu.sync_copy(rows_v.at[pl.ds(0, ln)], out_ref.at[pl.ds(off, ln)])


def _gs_pipeline(nch, table_c, gidx_v, sidx_v, bufs, gsems, ssems, acc,
                 hist_fn=None):
    nbuf = len(bufs)
    ngrp = nch // nbuf
    for j in range(nbuf):
        pltpu.async_copy(table_c.at[gidx_v.at[j]], bufs[j], gsems[j])

    def body(i, carry):
        for j in range(nbuf):
            c = nbuf * i + j
            pltpu.make_async_copy(
                table_c.at[gidx_v.at[c]], bufs[j], gsems[j]).wait()
            pltpu.async_copy(bufs[j], acc.at[sidx_v.at[c]], ssems[j],
                             add=True)
            if hist_fn is not None:
                hist_fn(c)

        @pl.when(i < ngrp - 1)
        def _():
            for j in range(nbuf):
                c = nbuf * i + j
                pltpu.make_async_copy(
                    bufs[j], acc.at[sidx_v.at[c]], ssems[j]).wait()
                pltpu.async_copy(table_c.at[gidx_v.at[c + nbuf]],
                                 bufs[j], gsems[j])
        return carry
    lax.fori_loop(0, ngrp, body, 0)

    for j in range(nbuf):
        c = nch - nbuf + j
        pltpu.make_async_copy(bufs[j], acc.at[sidx_v.at[c]], ssems[j]).wait()


def _hist_update(hist, idx_v, i):
    for j in range(K1 // 16):
        ids = idx_v[i, pl.ds(j * 16, 16)]
        cnts, last = plsc.scan_count(ids)
        plsc.addupdate_scatter(hist.at[0], [ids], cnts, mask=last)


def _sc_gs_counts_body(table, gidx, sidx,
                       out_sum, out_cnt_s, out_cnt_g,
                       gidx_v, sidx_v, b0, b1, b2, b3, b4, hist,
                       g0, g1, g2, g3, g4, s0, s1, s2, s3, s4,
                       acc):
    cid = lax.axis_index("c")
    sid = lax.axis_index("s")
    bufs = (b0, b1, b2, b3, b4)
    gsems = (g0, g1, g2, g3, g4)
    ssems = (s0, s1, s2, s3, s4)

    _zero_rows(b0)
    _zero_stripe(acc, sid * RPT, b0)
    zero16 = jnp.zeros((16,), jnp.int32)
    def zh(i, carry):
        hist[0, pl.ds(i * 16, 16)] = zero16
        return carry
    lax.fori_loop(0, NE // 16, zh, 0)
    pltpu.sync_copy(gidx.at[sid], gidx_v)
    pltpu.sync_copy(sidx.at[sid], sidx_v)
    plsc.subcore_barrier()

    def hist_fn(i):
        @pl.when(cid == 0)
        def _():
            _hist_update(hist, sidx_v, i)

        @pl.when(cid == 1)
        def _():
            _hist_update(hist, gidx_v, i)

    _gs_pipeline(NCH1, table.at[cid], gidx_v, sidx_v, bufs, gsems, ssems,
                 acc, hist_fn)

    plsc.subcore_barrier()
    _copy_out_stripe(acc, sid * RPT, b0, out_sum.at[cid, sid])

    @pl.when(cid == 0)
    def _():
        pltpu.sync_copy(hist, out_cnt_s.at[sid])

    @pl.when(cid == 1)
    def _():
        pltpu.sync_copy(hist, out_cnt_g.at[sid])


@functools.cache
def _sc_gs_counts():
    return pl.kernel(
        _sc_gs_counts_body,
        out_type=(
        jax.ShapeDtypeStruct((NC, NS, RPT, HC), jnp.float32),
        jax.ShapeDtypeStruct((NS, 1, NE), jnp.int32),
        jax.ShapeDtypeStruct((NS, 1, NV), jnp.int32),
    ),
    mesh=_mesh(),
    compiler_params=pltpu.CompilerParams(
        use_tc_tiling_on_sc=False, needs_layout_passes=False),
    scratch_types=(
        [pltpu.VMEM((NCH1, K1), jnp.int32)] * 2
        + [pltpu.VMEM((K1, HC), jnp.float32)] * NBUF1
        + [pltpu.VMEM((1, NE), jnp.int32)]
        + [pltpu.SemaphoreType.DMA] * (2 * NBUF1)
        + [pltpu.VMEM_SHARED((NV, HC), jnp.float32)]
    ),
)


def _sc_gs_body(table, gidx, sidx,
                out_sum,
                gidx_v, sidx_v, b0, b1, b2, b3, b4,
                g0, g1, g2, g3, g4, s0, s1, s2, s3, s4,
                acc):
    cid = lax.axis_index("c")
    sid = lax.axis_index("s")
    bufs = (b0, b1, b2, b3, b4)
    gsems = (g0, g1, g2, g3, g4)
    ssems = (s0, s1, s2, s3, s4)

    _zero_rows(b0)
    _zero_stripe(acc, sid * RPT, b0)
    pltpu.sync_copy(gidx.at[sid], gidx_v)
    pltpu.sync_copy(sidx.at[sid], sidx_v)
    plsc.subcore_barrier()

    _gs_pipeline(NCH2, table.at[cid], gidx_v, sidx_v, bufs, gsems, ssems,
                 acc)

    plsc.subcore_barrier()
    _copy_out_stripe(acc, sid * RPT, b0, out_sum.at[cid, sid])


@functools.cache
def _sc_gs():
    return pl.kernel(
        _sc_gs_body,
        out_type=jax.ShapeDtypeStruct((NC, NS, RPT, HC), jnp.float32),
    mesh=_mesh(),
    compiler_params=pltpu.CompilerParams(
        use_tc_tiling_on_sc=False, needs_layout_passes=False),
    scratch_types=(
        [pltpu.VMEM((NCH2, K2), jnp.int32)] * 2
        + [pltpu.VMEM((K2, HC), jnp.float32)] * NBUF2
        + [pltpu.SemaphoreType.DMA] * (2 * NBUF2)
        + [pltpu.VMEM_SHARED((NV, HC), jnp.float32)]
    ),
)


_BR = 2000


def _tc_linear_body(x_ref, w_ref, b_ref, o_ref):
    o_ref[0] = (
        jnp.dot(x_ref[...], w_ref[0], preferred_element_type=jnp.float32)
        + b_ref[0])


def _tc_linear(x, ws, bs):
    n = x.shape[0]
    return pl.pallas_call(
        _tc_linear_body,
        grid=(n // _BR, NC),
        in_specs=[
            pl.BlockSpec((_BR, C), lambda i, j: (i, 0)),
            pl.BlockSpec((1, C, HC), lambda i, j: (j, 0, 0)),
            pl.BlockSpec((1, 1, HC), lambda i, j: (j, 0, 0)),
        ],
        out_specs=pl.BlockSpec((1, _BR, HC), lambda i, j: (j, i, 0)),
        out_shape=jax.ShapeDtypeStruct((NC, n, HC), jnp.float32),
    )(x, ws, bs)


def _tc_count_sum_body(ce_ref, cv_ref, oe_ref, ov_ref):
    for c_ref, o_ref in ((ce_ref, oe_ref), (cv_ref, ov_ref)):
        tot = jnp.sum(c_ref[:, 0, :].astype(jnp.float32), axis=0)
        rec = 1.0 / jnp.maximum(tot.reshape(1, NE), 1.0)
        o_ref[...] = jnp.broadcast_to(jnp.transpose(rec), (NE, C))


def _tc_count_sum(ce, cv):
    return pl.pallas_call(
        _tc_count_sum_body,
        out_shape=(
            jax.ShapeDtypeStruct((NE, C), jnp.float32),
            jax.ShapeDtypeStruct((NV, C), jnp.float32),
        ),
    )(ce, cv)


def _tc_mean_relu_linear_body(s_ref, c_ref, w_ref, b_ref, o_ref):
    y = jnp.concatenate([s_ref[0], s_ref[1]], axis=1) * c_ref[...]
    y = jnp.maximum(y, 0.0)
    z = (jnp.dot(y, w_ref[...], preferred_element_type=jnp.float32)
         + b_ref[...])
    o_ref[...] = jnp.stack([z[:, :HC], z[:, HC:]])


def _tc_mean_relu_linear(s, c, w, b2):
    n = s.shape[1]
    return pl.pallas_call(
        _tc_mean_relu_linear_body,
        grid=(n // _BR,),
        in_specs=[
            pl.BlockSpec((NC, _BR, HC), lambda i: (0, i, 0)),
            pl.BlockSpec((_BR, C), lambda i: (i, 0)),
            pl.BlockSpec((C, C), lambda i: (0, 0)),
            pl.BlockSpec((1, C), lambda i: (0, 0)),
        ],
        out_specs=pl.BlockSpec((NC, _BR, HC), lambda i: (0, i, 0)),
        out_shape=jax.ShapeDtypeStruct((NC, n, HC), jnp.float32),
    )(s, c, w, b2)


def _tc_mean_relu_body(s_ref, c_ref, o_ref):
    y = jnp.concatenate([s_ref[0], s_ref[1]], axis=1) * c_ref[...]
    o_ref[...] = jnp.maximum(y, 0.0)


def _tc_mean_relu(s, c):
    n = s.shape[1]
    return pl.pallas_call(
        _tc_mean_relu_body,
        grid=(n // _BR,),
        in_specs=[
            pl.BlockSpec((NC, _BR, HC), lambda i: (0, i, 0)),
            pl.BlockSpec((_BR, C), lambda i: (i, 0)),
        ],
        out_specs=pl.BlockSpec((_BR, C), lambda i: (i, 0)),
        out_shape=jax.ShapeDtypeStruct((n, C), jnp.float32),
    )(s, c)


def kernel(X, Wv, bv, We, be, edge_index):
    v_idx = edge_index[0].astype(jnp.int32)
    e_idx = edge_index[1].astype(jnp.int32)
    v3a = v_idx.reshape(NS, NCH1, K1)
    e3a = e_idx.reshape(NS, NCH1, K1)
    v3b = v_idx.reshape(NS, NCH2, K2)
    e3b = e_idx.reshape(NS, NCH2, K2)

    wv_s = Wv.T.reshape(C, NC, HC).transpose(1, 0, 2)
    bv_s = bv.reshape(1, NC, HC).transpose(1, 0, 2)
    we_s = We.T.reshape(C, NC, HC).transpose(1, 0, 2)
    be_s = be.reshape(1, NC, HC).transpose(1, 0, 2)

    Xp = _tc_linear(X, wv_s, bv_s)
    ysum, cnt_e_p, cnt_v_p = _sc_gs_counts()(Xp, v3a, e3a)
    cnt_e, cnt_v = _tc_count_sum(cnt_e_p, cnt_v_p)
    Yp = _tc_mean_relu_linear(ysum.reshape(NC, NE, HC), cnt_e,
                              We.T, be.reshape(1, C))
    xsum = _sc_gs()(Yp, e3b, v3b).reshape(NC, NV, HC)
    return _tc_mean_relu(xsum, cnt_v)
